# baseline (speedup 1.0000x reference)
import jax
import jax.numpy as jnp
from jax.experimental import pallas as pl

_B = 16
_DW = 384
_DHC = 384 * 3
_MH = 512
_MWC = 512 * 3


def _pad_kernel(d_ref, o_ref):
    o_ref[0, :_DW, :_DHC] = d_ref[0]
    o_ref[0, :_DW, _DHC:] = jnp.full((_DW, _MWC - _DHC), jnp.nan, jnp.float32)
    o_ref[0, _DW:, :] = jnp.full((_MH - _DW, _MWC), jnp.nan, jnp.float32)


def kernel(data, shape):
    d2 = data.reshape(_B, _DW, _DHC)
    out = pl.pallas_call(
        _pad_kernel,
        grid=(_B,),
        in_specs=[pl.BlockSpec((1, _DW, _DHC), lambda b: (b, 0, 0))],
        out_specs=pl.BlockSpec((1, _MH, _MWC), lambda b: (b, 0, 0)),
        out_shape=jax.ShapeDtypeStruct((_B, _MH, _MWC), jnp.float32),
    )(d2)
    return out.reshape(_B, _MH, _MH, 3)
